# baseline (speedup 1.0000x reference)
import functools
import math

import jax
import jax.numpy as jnp
from jax import lax
from jax.experimental import pallas as pl
from jax.experimental.pallas import tpu as pltpu
from jax.experimental.pallas import tpu_sc as plsc

D_MODEL = 128
NUM_NODES = 10000
NUM_EDGES = 320000
SCALE = math.sqrt(float(D_MODEL))

NC = 2
NS = 16
NW = NC * NS

CH = 200

NODE_PAD = 12800
N_PER_W = NODE_PAD // NW
N_CHUNKS = N_PER_W // CH
E_PER_W = NUM_EDGES // NW
E_CHUNKS = E_PER_W // CH


EREP = 625


def _scale_body(ntab_ref, etab_ref, nout_ref, eout_ref):
    nout_ref[...] = ntab_ref[...] * SCALE
    eout_ref[...] = jnp.broadcast_to(etab_ref[...] * SCALE,
                                     (EREP, 16, D_MODEL))


def _scale_tables(node_table, edge_table):
    ntab_s, erep3 = pl.pallas_call(
        _scale_body,
        out_shape=(
            jax.ShapeDtypeStruct((NUM_NODES, D_MODEL), jnp.float32),
            jax.ShapeDtypeStruct((EREP, 16, D_MODEL), jnp.float32),
        ),
    )(node_table, edge_table)
    return ntab_s, erep3.reshape(EREP * 16, D_MODEL)


NBUF = 2


def _gather_body(nidx_hbm, eidx_hbm, ntab_hbm, etab_hbm,
                 nout_hbm, eout_hbm,
                 nidx_v, eidx_v, bufs_v, *sems):
    gsem = sems[:NBUF]
    ssem = sems[NBUF:]
    c = lax.axis_index("c")
    s = lax.axis_index("s")
    wid = s * NC + c

    nbase = wid * N_PER_W
    ebase = wid * E_PER_W

    pltpu.sync_copy(nidx_hbm.at[pl.ds(nbase, N_PER_W)], nidx_v)
    pltpu.sync_copy(eidx_hbm.at[pl.ds(ebase, E_PER_W)], eidx_v)

    lane = lax.iota(jnp.int32, 16)

    def spread(v, carry):
        off = pl.multiple_of(v * 16, 8)
        rep = (lane + v * 16) % EREP
        eidx_v[pl.ds(off, 16)] = eidx_v[pl.ds(off, 16)] + rep * 16
        return carry

    lax.fori_loop(0, E_PER_W // 16, spread, 0)

    def phase(idx_v, tab_hbm, out_hbm, base, chunks):
        ngroups = chunks // NBUF

        def gstart(b, off):
            pltpu.async_copy(tab_hbm.at[idx_v.at[pl.ds(off, CH)]],
                             bufs_v.at[b], gsem[b])

        def gwait(b):
            pltpu.make_async_copy(tab_hbm.at[idx_v.at[pl.ds(0, CH)]],
                                  bufs_v.at[b], gsem[b]).wait()

        def sstart(b, off):
            pltpu.async_copy(bufs_v.at[b], out_hbm.at[pl.ds(base + off, CH)],
                             ssem[b])

        def swait(b):
            pltpu.make_async_copy(bufs_v.at[b],
                                  out_hbm.at[pl.ds(base, CH)], ssem[b]).wait()

        for b in range(NBUF):
            gstart(b, b * CH)

        def group(g, carry):
            for b in range(NBUF):
                off = pl.multiple_of((g * NBUF + b) * CH, 8)
                gwait(b)
                sstart(b, off)
            for b in range(NBUF):
                nxt = pl.multiple_of(((g + 1) * NBUF + b) * CH, 8)

                @pl.when(g + 1 < ngroups)
                def _():
                    swait(b)
                    gstart(b, nxt)

            return carry

        lax.fori_loop(0, ngroups, group, 0)
        for b in range(NBUF):
            swait(b)

    phase(nidx_v, ntab_hbm, nout_hbm, nbase, N_CHUNKS)
    phase(eidx_v, etab_hbm, eout_hbm, ebase, E_CHUNKS)


def kernel(node_indices, edge_indices, edge_type_indices, node_table,
           edge_table):
    del edge_indices
    ntab_s, etab_s = _scale_tables(node_table, edge_table)

    nidx = jnp.zeros((NODE_PAD,), jnp.int32).at[:NUM_NODES].set(
        node_indices.astype(jnp.int32))
    eidx = edge_type_indices.astype(jnp.int32)

    mesh = plsc.VectorSubcoreMesh(core_axis_name="c", subcore_axis_name="s")
    gather = functools.partial(
        pl.kernel,
        mesh=mesh,
        out_type=(
            jax.ShapeDtypeStruct((NODE_PAD, D_MODEL), jnp.float32),
            jax.ShapeDtypeStruct((NUM_EDGES, D_MODEL), jnp.float32),
        ),
        scratch_types=(
            [
                pltpu.VMEM((N_PER_W,), jnp.int32),
                pltpu.VMEM((E_PER_W,), jnp.int32),
                pltpu.VMEM((NBUF, CH, D_MODEL), jnp.float32),
            ]
            + [pltpu.SemaphoreType.DMA] * (2 * NBUF)
        ),
    )(_gather_body)

    nout, eout = gather(nidx, eidx, ntab_s, etab_s)
    return (nout[:NUM_NODES], eout)


# speedup vs baseline: 1.0594x; 1.0594x over previous
import functools
import math

import jax
import jax.numpy as jnp
from jax import lax
from jax.experimental import pallas as pl
from jax.experimental.pallas import tpu as pltpu
from jax.experimental.pallas import tpu_sc as plsc

D_MODEL = 128
NUM_NODES = 10000
NUM_EDGES = 320000
SCALE = math.sqrt(float(D_MODEL))

NC = 2
NS = 16
NW = NC * NS

CH = 40

NODE_PAD = 12800
N_PER_W = NODE_PAD // NW
N_CHUNKS = N_PER_W // CH
E_PER_W = NUM_EDGES // NW
E_CHUNKS = E_PER_W // CH


EREP = 625


def _scale_body(ntab_ref, etab_ref, nout_ref, eout_ref):
    nout_ref[...] = ntab_ref[...] * SCALE
    eout_ref[...] = jnp.broadcast_to(etab_ref[...] * SCALE,
                                     (EREP, 16, D_MODEL))


def _scale_tables(node_table, edge_table):
    ntab_s, erep3 = pl.pallas_call(
        _scale_body,
        out_shape=(
            jax.ShapeDtypeStruct((NUM_NODES, D_MODEL), jnp.float32),
            jax.ShapeDtypeStruct((EREP, 16, D_MODEL), jnp.float32),
        ),
    )(node_table, edge_table)
    return ntab_s, erep3.reshape(EREP * 16, D_MODEL)


NBUF = 10


def _gather_body(nidx_hbm, eidx_hbm, ntab_hbm, etab_hbm,
                 nout_hbm, eout_hbm,
                 nidx_v, eidx_v, bufs_v, *sems):
    gsem = sems[:NBUF]
    ssem = sems[NBUF:]
    c = lax.axis_index("c")
    s = lax.axis_index("s")
    wid = s * NC + c

    nbase = wid * N_PER_W
    ebase = wid * E_PER_W

    pltpu.sync_copy(nidx_hbm.at[pl.ds(nbase, N_PER_W)], nidx_v)
    pltpu.sync_copy(eidx_hbm.at[pl.ds(ebase, E_PER_W)], eidx_v)

    lane = lax.iota(jnp.int32, 16)

    def spread(v, carry):
        off = pl.multiple_of(v * 16, 8)
        rep = (lane + v * 16) % EREP
        eidx_v[pl.ds(off, 16)] = eidx_v[pl.ds(off, 16)] + rep * 16
        return carry

    lax.fori_loop(0, E_PER_W // 16, spread, 0)

    def phase(idx_v, tab_hbm, out_hbm, base, chunks):
        ngroups = chunks // NBUF

        def gstart(b, off):
            pltpu.async_copy(tab_hbm.at[idx_v.at[pl.ds(off, CH)]],
                             bufs_v.at[b], gsem[b])

        def gwait(b):
            pltpu.make_async_copy(tab_hbm.at[idx_v.at[pl.ds(0, CH)]],
                                  bufs_v.at[b], gsem[b]).wait()

        def sstart(b, off):
            pltpu.async_copy(bufs_v.at[b], out_hbm.at[pl.ds(base + off, CH)],
                             ssem[b])

        def swait(b):
            pltpu.make_async_copy(bufs_v.at[b],
                                  out_hbm.at[pl.ds(base, CH)], ssem[b]).wait()

        for b in range(NBUF):
            gstart(b, b * CH)

        def group(g, carry):
            for b in range(NBUF):
                off = pl.multiple_of((g * NBUF + b) * CH, 8)
                gwait(b)
                sstart(b, off)
            for b in range(NBUF):
                nxt = pl.multiple_of(((g + 1) * NBUF + b) * CH, 8)

                @pl.when(g + 1 < ngroups)
                def _():
                    swait(b)
                    gstart(b, nxt)

            return carry

        lax.fori_loop(0, ngroups, group, 0)
        for b in range(NBUF):
            swait(b)

    phase(nidx_v, ntab_hbm, nout_hbm, nbase, N_CHUNKS)
    phase(eidx_v, etab_hbm, eout_hbm, ebase, E_CHUNKS)


def kernel(node_indices, edge_indices, edge_type_indices, node_table,
           edge_table):
    del edge_indices
    ntab_s, etab_s = _scale_tables(node_table, edge_table)

    nidx = jnp.zeros((NODE_PAD,), jnp.int32).at[:NUM_NODES].set(
        node_indices.astype(jnp.int32))
    eidx = edge_type_indices.astype(jnp.int32)

    mesh = plsc.VectorSubcoreMesh(core_axis_name="c", subcore_axis_name="s")
    gather = functools.partial(
        pl.kernel,
        mesh=mesh,
        out_type=(
            jax.ShapeDtypeStruct((NODE_PAD, D_MODEL), jnp.float32),
            jax.ShapeDtypeStruct((NUM_EDGES, D_MODEL), jnp.float32),
        ),
        scratch_types=(
            [
                pltpu.VMEM((N_PER_W,), jnp.int32),
                pltpu.VMEM((E_PER_W,), jnp.int32),
                pltpu.VMEM((NBUF, CH, D_MODEL), jnp.float32),
            ]
            + [pltpu.SemaphoreType.DMA] * (2 * NBUF)
        ),
    )(_gather_body)

    nout, eout = gather(nidx, eidx, ntab_s, etab_s)
    return (nout[:NUM_NODES], eout)


# speedup vs baseline: 1.5864x; 1.4975x over previous
import functools
import math

import jax
import jax.numpy as jnp
from jax import lax
from jax.experimental import pallas as pl
from jax.experimental.pallas import tpu as pltpu
from jax.experimental.pallas import tpu_sc as plsc

D_MODEL = 128
NUM_NODES = 10000
NUM_EDGES = 320000
SCALE = math.sqrt(float(D_MODEL))

NC = 2
NS = 16
NW = NC * NS

CH = 40

NODE_PAD = 12800
N_PER_W = NODE_PAD // NW
N_CHUNKS = N_PER_W // CH
E_PER_W = NUM_EDGES // NW
E_CHUNKS = E_PER_W // CH


def _scale_body(ntab_ref, etab_ref, nout_ref, eout_ref):
    nout_ref[...] = ntab_ref[...] * SCALE
    eout_ref[...] = etab_ref[...] * SCALE


def _scale_tables(node_table, edge_table):
    return pl.pallas_call(
        _scale_body,
        out_shape=(
            jax.ShapeDtypeStruct((NUM_NODES, D_MODEL), jnp.float32),
            jax.ShapeDtypeStruct((16, D_MODEL), jnp.float32),
        ),
    )(node_table, edge_table)


NBUF = 10


def _gather_body(nidx_hbm, eidx_hbm, ntab_hbm, etab_hbm,
                 nout_hbm, eout_hbm,
                 nidx_v, eidx_v, etab_v, bufs_v, *sems):
    gsem = sems[:NBUF]
    ssem = sems[NBUF:]
    c = lax.axis_index("c")
    s = lax.axis_index("s")
    wid = s * NC + c

    nbase = wid * N_PER_W
    ebase = wid * E_PER_W

    pltpu.sync_copy(nidx_hbm.at[pl.ds(nbase, N_PER_W)], nidx_v)
    pltpu.sync_copy(eidx_hbm.at[pl.ds(ebase, E_PER_W)], eidx_v)
    @pl.when(s == 0)
    def _():
        pltpu.sync_copy(etab_hbm, etab_v)

    plsc.subcore_barrier()

    def phase(idx_v, tab_hbm, out_hbm, base, chunks):
        ngroups = chunks // NBUF

        def gstart(b, off):
            pltpu.async_copy(tab_hbm.at[idx_v.at[pl.ds(off, CH)]],
                             bufs_v.at[b], gsem[b])

        def gwait(b):
            pltpu.make_async_copy(tab_hbm.at[idx_v.at[pl.ds(0, CH)]],
                                  bufs_v.at[b], gsem[b]).wait()

        def sstart(b, off):
            pltpu.async_copy(bufs_v.at[b], out_hbm.at[pl.ds(base + off, CH)],
                             ssem[b])

        def swait(b):
            pltpu.make_async_copy(bufs_v.at[b],
                                  out_hbm.at[pl.ds(base, CH)], ssem[b]).wait()

        for b in range(NBUF):
            gstart(b, b * CH)

        def group(g, carry):
            for b in range(NBUF):
                off = pl.multiple_of((g * NBUF + b) * CH, 8)
                gwait(b)
                sstart(b, off)
            for b in range(NBUF):
                nxt = pl.multiple_of(((g + 1) * NBUF + b) * CH, 8)

                @pl.when(g + 1 < ngroups)
                def _():
                    swait(b)
                    gstart(b, nxt)

            return carry

        lax.fori_loop(0, ngroups, group, 0)
        for b in range(NBUF):
            swait(b)

    phase(nidx_v, ntab_hbm, nout_hbm, nbase, N_CHUNKS)
    phase(eidx_v, etab_v, eout_hbm, ebase, E_CHUNKS)


def kernel(node_indices, edge_indices, edge_type_indices, node_table,
           edge_table):
    del edge_indices
    ntab_s, etab_s = _scale_tables(node_table, edge_table)

    nidx = jnp.zeros((NODE_PAD,), jnp.int32).at[:NUM_NODES].set(
        node_indices.astype(jnp.int32))
    eidx = edge_type_indices.astype(jnp.int32)

    mesh = plsc.VectorSubcoreMesh(core_axis_name="c", subcore_axis_name="s")
    gather = functools.partial(
        pl.kernel,
        mesh=mesh,
        out_type=(
            jax.ShapeDtypeStruct((NODE_PAD, D_MODEL), jnp.float32),
            jax.ShapeDtypeStruct((NUM_EDGES, D_MODEL), jnp.float32),
        ),
        scratch_types=(
            [
                pltpu.VMEM((N_PER_W,), jnp.int32),
                pltpu.VMEM((E_PER_W,), jnp.int32),
                pltpu.VMEM_SHARED((16, D_MODEL), jnp.float32),
                pltpu.VMEM((NBUF, CH, D_MODEL), jnp.float32),
            ]
            + [pltpu.SemaphoreType.DMA] * (2 * NBUF)
        ),
    )(_gather_body)

    nout, eout = gather(nidx, eidx, ntab_s, etab_s)
    return (nout[:NUM_NODES], eout)
